# baseline (speedup 1.0000x reference)
import functools

import jax
import jax.numpy as jnp
from jax import lax
from jax.experimental import pallas as pl
from jax.experimental.pallas import tpu as pltpu
from jax.experimental.pallas import tpu_sc as plsc

_N = 1024
_C = 128
_K = 18
_F32 = jnp.float32

_GX = (-0.2, -0.2, 0.2, 0.2)
_GY = (-0.2, 0.2, -0.2, 0.2)

_NC = 2
_NS = 16
_NW = _NC * _NS
_PPW = _N // _NW
_PCH = 8
_NCH = _PPW // _PCH


def _stage1_body(xt_ref, x_ref, w0u_ref, w0y_ref, w1u_ref, w1y_ref,
                 ig0_ref, ig1_ref, u0_ref, y0_ref, u1_ref, y1_ref):
    xt = xt_ref[...]
    x = x_ref[...]
    g = jnp.dot(xt, x, preferred_element_type=_F32)
    sqc = jnp.sum(xt * xt, axis=1, keepdims=True)
    sqr = jnp.sum(x * x, axis=0, keepdims=True)
    dist = sqc - 2.0 * g + sqr
    col = jax.lax.broadcasted_iota(jnp.int32, (_N, _N), 1).astype(_F32)
    nblk = 8
    bw = _N // nblk
    for t in range(_K):
        vb = dist[:, 0:bw]
        ib = col[:, 0:bw]
        for blk in range(1, nblk):
            nv = dist[:, blk * bw:(blk + 1) * bw]
            keep = vb <= nv
            vb = jnp.where(keep, vb, nv)
            ib = jnp.where(keep, ib, col[:, blk * bw:(blk + 1) * bw])
        rowmin = jnp.min(vb, axis=1, keepdims=True)
        candl = jnp.where(vb <= rowmin, ib, _F32(2.0 * _N))
        amin = jnp.min(candl, axis=1, keepdims=True)
        gi = amin.astype(jnp.int32)
        if t < 9:
            ig0_ref[:, t:t + 1] = gi
        if t % 2 == 0:
            ig1_ref[:, t // 2:t // 2 + 1] = gi
        dist = jnp.where(col == amin, _F32(jnp.inf), dist)
    u0_ref[...] = jnp.dot(xt, w0u_ref[...], preferred_element_type=_F32)
    y0_ref[...] = jnp.dot(xt, w0y_ref[...], preferred_element_type=_F32)
    u1_ref[...] = jnp.dot(xt, w1u_ref[...], preferred_element_type=_F32)
    y1_ref[...] = jnp.dot(xt, w1y_ref[...], preferred_element_type=_F32)


def _sc_gather_body(y0_hbm, y1_hbm, u0_hbm, u1_hbm, i0_hbm, i1_hbm,
                    mx_hbm, st_hbm,
                    i0_v, i1_v, u0_v, u1_v, r0a_v, r1a_v, r0b_v, r1b_v,
                    o_v, a_v, sem_a, sem_b, sem_i):
    wid = lax.axis_index("s") * _NC + lax.axis_index("c")
    base = wid * _PPW
    pltpu.async_copy(i0_hbm.at[pl.ds(base * 9, _PPW * 9)], i0_v, sem_i).wait()
    pltpu.async_copy(i1_hbm.at[pl.ds(base * 9, _PPW * 9)], i1_v, sem_i).wait()
    pltpu.async_copy(u0_hbm.at[pl.ds(base, _PPW)], u0_v, sem_i).wait()
    pltpu.async_copy(u1_hbm.at[pl.ds(base, _PPW)], u1_v, sem_i).wait()

    def zero_body(i, carry):
        z = jnp.zeros((16,), _F32)
        for j in range(4):
            a_v[j, pl.ds(i * 16, 16)] = z
        return carry

    lax.fori_loop(0, 16, zero_body, 0)

    bufs = ((r0a_v, r1a_v, sem_a), (r0b_v, r1b_v, sem_b))

    def issue(c):
        r0_v, r1_v, sem = bufs[c % 2]
        sl = pl.ds(c * _PCH * 9, _PCH * 9)
        h0 = pltpu.async_copy(y0_hbm.at[i0_v.at[sl]], r0_v, sem)
        h1 = pltpu.async_copy(y1_hbm.at[i1_v.at[sl]], r1_v, sem)
        return h0, h1

    pending = issue(0)
    for c in range(_NCH):
        nxt = issue(c + 1) if c + 1 < _NCH else None
        pending[0].wait()
        pending[1].wait()
        r0_v, r1_v, _ = bufs[c % 2]
        coff = c * _PCH

        def point_body(p, carry, r0_v=r0_v, r1_v=r1_v, coff=coff):
            def tree(op, xs):
                xs = list(xs)
                while len(xs) > 1:
                    ys = [op(xs[i], xs[i + 1])
                          for i in range(0, len(xs) - 1, 2)]
                    if len(xs) % 2:
                        ys.append(xs[-1])
                    xs = ys
                return xs[0]

            for rows, u_v, ooff, aoff in ((r0_v, u0_v, 0, 0),
                                          (r1_v, u1_v, 1, 2)):
                for c16 in range(16):
                    sl = pl.ds(c16 * 16, 16)
                    u = u_v[coff + p, sl]
                    t = [u + rows[p * 9 + k, sl] for k in range(9)]
                    o_v[ooff, p, sl] = tree(jnp.maximum, t)
                    a_v[aoff, sl] = a_v[aoff, sl] + tree(jnp.add, t)
                    a_v[aoff + 1, sl] = (a_v[aoff + 1, sl]
                                         + tree(jnp.add, [x * x for x in t]))
            return carry

        lax.fori_loop(0, _PCH, point_body, 0)
        pltpu.sync_copy(o_v, mx_hbm.at[:, pl.ds(base + coff, _PCH)])
        pending = nxt

    pltpu.sync_copy(a_v, st_hbm.at[wid])


@functools.lru_cache(maxsize=1)
def _sc_gather_kernel():
    return pl.kernel(
        _sc_gather_body,
        mesh=plsc.VectorSubcoreMesh(core_axis_name="c", subcore_axis_name="s"),
        out_type=[jax.ShapeDtypeStruct((2, _N, 2 * _C), _F32),
                  jax.ShapeDtypeStruct((_NW, 4, 2 * _C), _F32)],
        scratch_types=[
            pltpu.VMEM((_PPW * 9,), jnp.int32),
            pltpu.VMEM((_PPW * 9,), jnp.int32),
            pltpu.VMEM((_PPW, 2 * _C), _F32),
            pltpu.VMEM((_PPW, 2 * _C), _F32),
            pltpu.VMEM((_PCH * 9, 2 * _C), _F32),
            pltpu.VMEM((_PCH * 9, 2 * _C), _F32),
            pltpu.VMEM((_PCH * 9, 2 * _C), _F32),
            pltpu.VMEM((_PCH * 9, 2 * _C), _F32),
            pltpu.VMEM((2, _PCH, 2 * _C), _F32),
            pltpu.VMEM((4, 2 * _C), _F32),
            pltpu.SemaphoreType.DMA,
            pltpu.SemaphoreType.DMA,
            pltpu.SemaphoreType.DMA,
        ],
    )


def _sc_gather_call(y0f, y1f, u0, u1, i0f, i1f):
    return _sc_gather_kernel()(y0f, y1f, u0, u1, i0f, i1f)


def _stage3_body(mx_0, st_0, mx_1, st_1,
                 g0_ref, b0_ref, g1_ref, b1n_ref,
                 w1a_ref, w1g_ref, b1_ref, w2_ref, b2_ref,
                 out_ref):
    nb = 2
    sc_refs = (mx_0, mx_1)
    st_refs = (st_0, st_1)
    cnt = _F32(nb * _N * 9)

    def bn_affine(si, g_ref, b_ref):
        s1 = jnp.zeros((1, 2 * _C), _F32)
        s2 = jnp.zeros((1, 2 * _C), _F32)
        for b in range(nb):
            s1 = s1 + jnp.sum(st_refs[b][:, si, :], axis=0, keepdims=True)
            s2 = s2 + jnp.sum(st_refs[b][:, si + 1, :], axis=0, keepdims=True)
        mean = s1 / cnt
        var = s2 / cnt - mean * mean
        scale = g_ref[...] * jax.lax.rsqrt(var + 1e-5)
        shift = b_ref[...] - mean * scale
        return scale, shift

    sc0, sh0 = bn_affine(0, g0_ref, b0_ref)
    sc1, sh1 = bn_affine(2, g1_ref, b1n_ref)

    w1a = w1a_ref[...]
    w2t = w2_ref[...]
    b2r = b2_ref[...]
    cvec = [b1_ref[...] + _GX[q] * w1g_ref[0:1, :] + _GY[q] * w1g_ref[1:2, :]
            for q in range(4)]

    for b in range(nb):
        x1n = jax.nn.relu(sc_refs[b][0] * sc0 + sh0)
        x2n = jax.nn.relu(sc_refs[b][1] * sc1 + sh1)
        feats = (x1n[:, :_C], x1n[:, _C:], x2n[:, :_C], x2n[:, _C:])
        for j in range(4):
            t = jnp.dot(feats[j], w1a, preferred_element_type=_F32)
            q = _N // 4
            aj = jnp.concatenate(
                [jax.nn.relu(t[i * q:(i + 1) * q, :] + cvec[i])
                 for i in range(4)], axis=0)
            out_ref[b, j] = jax.nn.relu(
                jnp.dot(aj, w2t, preferred_element_type=_F32) + b2r)


@jax.jit
def _run(x, w0u, w0y, w1u, w1y, bn0g, bn0b, bn1g, bn1b, w1a, w1g, b1, w2t, b2):
    nb = x.shape[0]
    xt = jnp.transpose(x, (0, 2, 1))

    stage3_in = []
    for b in range(nb):
        ig0, ig1, u0, y0, u1, y1 = pl.pallas_call(
            _stage1_body,
            out_shape=[jax.ShapeDtypeStruct((_N, 16), jnp.int32)] * 2
            + [jax.ShapeDtypeStruct((_N, 2 * _C), _F32)] * 4,
        )(xt[b], x[b], w0u, w0y, w1u, w1y)
        i0f = ig0[:, :9].reshape(-1)
        i1f = ig1[:, :9].reshape(-1)
        mxb, stb = _sc_gather_call(y0, y1, u0, u1, i0f, i1f)
        stage3_in += [mxb, stb]

    res = pl.pallas_call(
        _stage3_body,
        out_shape=jax.ShapeDtypeStruct((nb, 4, _N, _C), _F32),
    )(*stage3_in,
      bn0g.reshape(1, -1), bn0b.reshape(1, -1),
      bn1g.reshape(1, -1), bn1b.reshape(1, -1),
      w1a, w1g, b1.reshape(1, -1), w2t, b2.reshape(1, -1))

    return jnp.reshape(jnp.transpose(res, (0, 3, 2, 1)), (nb, _C, 4 * _N))


def kernel(input, W_dc0, bn0g, bn0b, W_dc1, bn1g, bn1b,
           WF, bF, bnFg, bnFb, WG, bG, bnGg, bnGb, WH, bH, bnHg, bnHb,
           gamma_ra, W1, b1, W2, b2):
    w0i, w0j = W_dc0[:, :_C], W_dc0[:, _C:]
    w1i, w1j = W_dc1[:, :_C], W_dc1[:, _C:]
    return _run(input,
                (w0i - w0j).T, w0j.T, (w1i - w1j).T, w1j.T,
                bn0g, bn0b, bn1g, bn1b,
                W1[:, :_C].T, W1[:, _C:_C + 2].T, b1, W2.T, b2)
